# baseline (speedup 1.0000x reference)
import jax
import jax.numpy as jnp
from jax import lax
from jax.experimental import pallas as pl
from jax.experimental.pallas import tpu as pltpu
from jax.experimental.pallas import tpu_sc as plsc

N = 10000
D = 128
E = 320000

NC = 2
NS = 16
NW = NC * NS
CH = 128
K = -(-(E // NW) // CH)
KA = 60
KB = 97
KMAX = KB
EA = NS * KA * CH
EPB = -(-(E - EA) // NS)
EPW = K * CH
E_PAD = EPW * NW

RPT = -(-(N + 1) // (NS * 8)) * 8
N_ACC = RPT * NS

_MESH = plsc.VectorSubcoreMesh(
    core_axis_name="c", subcore_axis_name="s", num_cores=NC, num_subcores=NS
)


def _agg_body(hn_hbm, srcw, dstw, zacc, parts, src_v, dst_v, rows_v, acc_sh,
              sem):
    c = lax.axis_index("c")
    s = lax.axis_index("s")
    wid = c * NS + s
    r0 = s * RPT
    pltpu.sync_copy(zacc.at[pl.ds(r0, RPT)], acc_sh.at[pl.ds(r0, RPT)])
    pltpu.sync_copy(srcw.at[wid], src_v)
    pltpu.sync_copy(dstw.at[wid], dst_v)
    plsc.subcore_barrier()

    def step(j, carry):
        pltpu.async_copy(hn_hbm.at[src_v.at[j]], rows_v, sem).wait()
        pltpu.sync_copy(rows_v, acc_sh.at[dst_v.at[j]], add=True)
        return carry

    @pl.when(c == 0)
    def _():
        lax.fori_loop(0, KA, step, 0)

    @pl.when(c == 1)
    def _():
        lax.fori_loop(0, KB, step, 0)

    plsc.subcore_barrier()
    pltpu.sync_copy(acc_sh.at[pl.ds(r0, RPT)], parts.at[c, pl.ds(r0, RPT)])


_agg = pl.kernel(
    _agg_body,
    out_type=jax.ShapeDtypeStruct((NC, N_ACC, D), jnp.float32),
    mesh=_MESH,
    scratch_types=[
        pltpu.VMEM((KMAX, CH), jnp.int32),
        pltpu.VMEM((KMAX, CH), jnp.int32),
        pltpu.VMEM((CH, D), jnp.float32),
        pltpu.VMEM_SHARED((N_ACC, D), jnp.float32),
        pltpu.SemaphoreType.DMA,
    ],
)


def _deg_body(dstw, zdeg, ones_hbm, degp, dst_v, ones_v, deg_sh):
    c = lax.axis_index("c")
    s = lax.axis_index("s")
    wid = c * NS + s
    r0 = s * RPT
    pltpu.sync_copy(zdeg.at[pl.ds(r0, RPT)], deg_sh.at[pl.ds(r0, RPT)])
    pltpu.sync_copy(dstw.at[wid], dst_v)
    pltpu.sync_copy(ones_hbm, ones_v)
    plsc.subcore_barrier()

    def step(j, carry):
        pltpu.sync_copy(ones_v, deg_sh.at[dst_v.at[j]], add=True)
        return carry

    lax.fori_loop(0, K, step, 0)
    plsc.subcore_barrier()
    pltpu.sync_copy(deg_sh.at[pl.ds(r0, RPT)], degp.at[c, pl.ds(r0, RPT)])


_deg = pl.kernel(
    _deg_body,
    out_type=jax.ShapeDtypeStruct((NC, N_ACC, D), jnp.float32),
    mesh=_MESH,
    scratch_types=[
        pltpu.VMEM((K, CH), jnp.int32),
        pltpu.VMEM((CH, D), jnp.float32),
        pltpu.VMEM_SHARED((N_ACC, D), jnp.float32),
    ],
)


BM = 400


def _lin2_tc(h_ref, ws_ref, wn_ref, b_ref, hs_ref, hn_ref):
    h = h_ref[...]
    hs_ref[...] = (
        jnp.dot(h, ws_ref[...], preferred_element_type=jnp.float32) + b_ref[...]
    )
    hn_ref[...] = jnp.dot(h, wn_ref[...], preferred_element_type=jnp.float32)


def _lin2(h, Ws, Wn, b):
    return pl.pallas_call(
        _lin2_tc,
        grid=(N // BM,),
        in_specs=[
            pl.BlockSpec((BM, D), lambda i: (i, 0)),
            pl.BlockSpec((D, D), lambda i: (0, 0)),
            pl.BlockSpec((D, D), lambda i: (0, 0)),
            pl.BlockSpec((1, D), lambda i: (0, 0)),
        ],
        out_specs=[
            pl.BlockSpec((BM, D), lambda i: (i, 0)),
            pl.BlockSpec((BM, D), lambda i: (i, 0)),
        ],
        out_shape=[jax.ShapeDtypeStruct((N, D), jnp.float32)] * 2,
    )(h, Ws, Wn, b.reshape(1, D))


def _mean_relu(hs, p0, p1, d0, d1):
    deg = jnp.maximum((d0 + d1)[:, 0:1], 1.0)
    return jnp.maximum(hs + (p0 + p1) / deg, 0.0)


def _comb_lin2_tc(hs_ref, p0_ref, p1_ref, d0_ref, d1_ref, ws_ref, wn_ref, b_ref,
                  h1_ref, hs2_ref, hn2_ref):
    h1 = _mean_relu(hs_ref[...], p0_ref[...], p1_ref[...], d0_ref[...], d1_ref[...])
    h1_ref[...] = h1
    hs2_ref[...] = (
        jnp.dot(h1, ws_ref[...], preferred_element_type=jnp.float32) + b_ref[...]
    )
    hn2_ref[...] = jnp.dot(h1, wn_ref[...], preferred_element_type=jnp.float32)


def _comb_lin2(hs, p0, p1, d0, d1, Ws, Wn, b):
    return pl.pallas_call(
        _comb_lin2_tc,
        grid=(N // BM,),
        in_specs=[
            pl.BlockSpec((BM, D), lambda i: (i, 0)),
            pl.BlockSpec((BM, D), lambda i: (i, 0)),
            pl.BlockSpec((BM, D), lambda i: (i, 0)),
            pl.BlockSpec((BM, D), lambda i: (i, 0)),
            pl.BlockSpec((BM, D), lambda i: (i, 0)),
            pl.BlockSpec((D, D), lambda i: (0, 0)),
            pl.BlockSpec((D, D), lambda i: (0, 0)),
            pl.BlockSpec((1, D), lambda i: (0, 0)),
        ],
        out_specs=[
            pl.BlockSpec((BM, D), lambda i: (i, 0)),
            pl.BlockSpec((BM, D), lambda i: (i, 0)),
            pl.BlockSpec((BM, D), lambda i: (i, 0)),
        ],
        out_shape=[jax.ShapeDtypeStruct((N, D), jnp.float32)] * 3,
    )(hs, p0, p1, d0, d1, Ws, Wn, b.reshape(1, D))


def _comb_tc(hs_ref, p0_ref, p1_ref, d0_ref, d1_ref, out_ref):
    out_ref[...] = _mean_relu(
        hs_ref[...], p0_ref[...], p1_ref[...], d0_ref[...], d1_ref[...]
    )


def _comb(hs, p0, p1, d0, d1):
    return pl.pallas_call(
        _comb_tc,
        grid=(N // BM,),
        in_specs=[
            pl.BlockSpec((BM, D), lambda i: (i, 0)),
            pl.BlockSpec((BM, D), lambda i: (i, 0)),
            pl.BlockSpec((BM, D), lambda i: (i, 0)),
            pl.BlockSpec((BM, D), lambda i: (i, 0)),
            pl.BlockSpec((BM, D), lambda i: (i, 0)),
        ],
        out_specs=pl.BlockSpec((BM, D), lambda i: (i, 0)),
        out_shape=jax.ShapeDtypeStruct((N, D), jnp.float32),
    )(hs, p0, p1, d0, d1)


def kernel(x, edge_index, W1_self, W1_nbr, b1, W2_self, W2_nbr, b2):
    src = edge_index[0].astype(jnp.int32)
    dst = edge_index[1].astype(jnp.int32)

    def _split(v, fill):
        a = jnp.pad(
            v[:EA].reshape(NS, KA, CH),
            ((0, 0), (0, KMAX - KA), (0, 0)),
            constant_values=fill,
        )
        b = jnp.pad(
            v[EA:].reshape(NS, EPB),
            ((0, 0), (0, KMAX * CH - EPB)),
            constant_values=fill,
        ).reshape(NS, KMAX, CH)
        return jnp.concatenate([a, b], axis=0)

    srcw = _split(src, 0)
    dstw = _split(dst, N)
    padd = E_PAD - E
    dstwd = jnp.concatenate([dst, jnp.full((padd,), N, jnp.int32)]).reshape(
        NW, K, CH
    )
    zacc = jnp.zeros((N_ACC, D), jnp.float32)
    ones = jnp.ones((CH, D), jnp.float32)

    hs1, hn1 = _lin2(x, W1_self, W1_nbr, b1)
    degp = _deg(dstwd, zacc, ones)
    hn1, degp = lax.optimization_barrier((hn1, degp))
    parts1 = _agg(hn1, srcw, dstw, zacc)
    d0, d1 = degp[0, :N], degp[1, :N]
    h1, hs2, hn2 = _comb_lin2(
        hs1, parts1[0, :N], parts1[1, :N], d0, d1, W2_self, W2_nbr, b2
    )
    parts2 = _agg(hn2, srcw, dstw, zacc)
    h2 = _comb(hs2, parts2[0, :N], parts2[1, :N], d0, d1)
    return (x, h1, h2)


# speedup vs baseline: 1.0831x; 1.0831x over previous
import jax
import jax.numpy as jnp
from jax import lax
from jax.experimental import pallas as pl
from jax.experimental.pallas import tpu as pltpu
from jax.experimental.pallas import tpu_sc as plsc

N = 10000
D = 128
E = 320000

NC = 2
NS = 16
NW = NC * NS
CH = 128
K = -(-(E // NW) // CH)
KA = 75
KB = 82
KMAX = KB
EA = NS * KA * CH
EPB = -(-(E - EA) // NS)
EPW = K * CH
E_PAD = EPW * NW

RPT = -(-(N + 1) // (NS * 8)) * 8
N_ACC = RPT * NS

_MESH = plsc.VectorSubcoreMesh(
    core_axis_name="c", subcore_axis_name="s", num_cores=NC, num_subcores=NS
)


def _agg_body(hn_hbm, srcw, dstw, zacc, parts, src_v, dst_v, rows_v, acc_sh,
              sem):
    c = lax.axis_index("c")
    s = lax.axis_index("s")
    wid = c * NS + s
    r0 = s * RPT
    pltpu.sync_copy(zacc.at[pl.ds(r0, RPT)], acc_sh.at[pl.ds(r0, RPT)])
    pltpu.sync_copy(srcw.at[wid], src_v)
    pltpu.sync_copy(dstw.at[wid], dst_v)
    plsc.subcore_barrier()

    def step(j, carry):
        pltpu.async_copy(hn_hbm.at[src_v.at[j]], rows_v, sem).wait()
        pltpu.sync_copy(rows_v, acc_sh.at[dst_v.at[j]], add=True)
        return carry

    @pl.when(c == 0)
    def _():
        lax.fori_loop(0, KA, step, 0)

    @pl.when(c == 1)
    def _():
        lax.fori_loop(0, KB, step, 0)

    plsc.subcore_barrier()
    pltpu.sync_copy(acc_sh.at[pl.ds(r0, RPT)], parts.at[c, pl.ds(r0, RPT)])


_agg = pl.kernel(
    _agg_body,
    out_type=jax.ShapeDtypeStruct((NC, N_ACC, D), jnp.float32),
    mesh=_MESH,
    scratch_types=[
        pltpu.VMEM((KMAX, CH), jnp.int32),
        pltpu.VMEM((KMAX, CH), jnp.int32),
        pltpu.VMEM((CH, D), jnp.float32),
        pltpu.VMEM_SHARED((N_ACC, D), jnp.float32),
        pltpu.SemaphoreType.DMA,
    ],
)


def _deg_body(dstw, zdeg, ones_hbm, degp, dst_v, ones_v, deg_sh):
    c = lax.axis_index("c")
    s = lax.axis_index("s")
    wid = c * NS + s
    r0 = s * RPT
    pltpu.sync_copy(zdeg.at[pl.ds(r0, RPT)], deg_sh.at[pl.ds(r0, RPT)])
    pltpu.sync_copy(dstw.at[wid], dst_v)
    pltpu.sync_copy(ones_hbm, ones_v)
    plsc.subcore_barrier()

    def step(j, carry):
        pltpu.sync_copy(ones_v, deg_sh.at[dst_v.at[j]], add=True)
        return carry

    lax.fori_loop(0, K, step, 0)
    plsc.subcore_barrier()
    pltpu.sync_copy(deg_sh.at[pl.ds(r0, RPT)], degp.at[c, pl.ds(r0, RPT)])


_deg = pl.kernel(
    _deg_body,
    out_type=jax.ShapeDtypeStruct((NC, N_ACC, D), jnp.float32),
    mesh=_MESH,
    scratch_types=[
        pltpu.VMEM((K, CH), jnp.int32),
        pltpu.VMEM((CH, D), jnp.float32),
        pltpu.VMEM_SHARED((N_ACC, D), jnp.float32),
    ],
)


BM = 400


def _lin2_tc(h_ref, ws_ref, wn_ref, b_ref, hs_ref, hn_ref):
    h = h_ref[...]
    hs_ref[...] = (
        jnp.dot(h, ws_ref[...], preferred_element_type=jnp.float32) + b_ref[...]
    )
    hn_ref[...] = jnp.dot(h, wn_ref[...], preferred_element_type=jnp.float32)


def _lin2(h, Ws, Wn, b):
    return pl.pallas_call(
        _lin2_tc,
        grid=(N // BM,),
        in_specs=[
            pl.BlockSpec((BM, D), lambda i: (i, 0)),
            pl.BlockSpec((D, D), lambda i: (0, 0)),
            pl.BlockSpec((D, D), lambda i: (0, 0)),
            pl.BlockSpec((1, D), lambda i: (0, 0)),
        ],
        out_specs=[
            pl.BlockSpec((BM, D), lambda i: (i, 0)),
            pl.BlockSpec((BM, D), lambda i: (i, 0)),
        ],
        out_shape=[jax.ShapeDtypeStruct((N, D), jnp.float32)] * 2,
    )(h, Ws, Wn, b.reshape(1, D))


def _mean_relu(hs, p0, p1, d0, d1):
    deg = jnp.maximum((d0 + d1)[:, 0:1], 1.0)
    return jnp.maximum(hs + (p0 + p1) / deg, 0.0)


def _comb_lin2_tc(hs_ref, p0_ref, p1_ref, d0_ref, d1_ref, ws_ref, wn_ref, b_ref,
                  h1_ref, hs2_ref, hn2_ref):
    h1 = _mean_relu(hs_ref[...], p0_ref[...], p1_ref[...], d0_ref[...], d1_ref[...])
    h1_ref[...] = h1
    hs2_ref[...] = (
        jnp.dot(h1, ws_ref[...], preferred_element_type=jnp.float32) + b_ref[...]
    )
    hn2_ref[...] = jnp.dot(h1, wn_ref[...], preferred_element_type=jnp.float32)


def _comb_lin2(hs, p0, p1, d0, d1, Ws, Wn, b):
    return pl.pallas_call(
        _comb_lin2_tc,
        grid=(N // BM,),
        in_specs=[
            pl.BlockSpec((BM, D), lambda i: (i, 0)),
            pl.BlockSpec((BM, D), lambda i: (i, 0)),
            pl.BlockSpec((BM, D), lambda i: (i, 0)),
            pl.BlockSpec((BM, D), lambda i: (i, 0)),
            pl.BlockSpec((BM, D), lambda i: (i, 0)),
            pl.BlockSpec((D, D), lambda i: (0, 0)),
            pl.BlockSpec((D, D), lambda i: (0, 0)),
            pl.BlockSpec((1, D), lambda i: (0, 0)),
        ],
        out_specs=[
            pl.BlockSpec((BM, D), lambda i: (i, 0)),
            pl.BlockSpec((BM, D), lambda i: (i, 0)),
            pl.BlockSpec((BM, D), lambda i: (i, 0)),
        ],
        out_shape=[jax.ShapeDtypeStruct((N, D), jnp.float32)] * 3,
    )(hs, p0, p1, d0, d1, Ws, Wn, b.reshape(1, D))


def _comb_tc(hs_ref, p0_ref, p1_ref, d0_ref, d1_ref, out_ref):
    out_ref[...] = _mean_relu(
        hs_ref[...], p0_ref[...], p1_ref[...], d0_ref[...], d1_ref[...]
    )


def _comb(hs, p0, p1, d0, d1):
    return pl.pallas_call(
        _comb_tc,
        grid=(N // BM,),
        in_specs=[
            pl.BlockSpec((BM, D), lambda i: (i, 0)),
            pl.BlockSpec((BM, D), lambda i: (i, 0)),
            pl.BlockSpec((BM, D), lambda i: (i, 0)),
            pl.BlockSpec((BM, D), lambda i: (i, 0)),
            pl.BlockSpec((BM, D), lambda i: (i, 0)),
        ],
        out_specs=pl.BlockSpec((BM, D), lambda i: (i, 0)),
        out_shape=jax.ShapeDtypeStruct((N, D), jnp.float32),
    )(hs, p0, p1, d0, d1)


def kernel(x, edge_index, W1_self, W1_nbr, b1, W2_self, W2_nbr, b2):
    src = edge_index[0].astype(jnp.int32)
    dst = edge_index[1].astype(jnp.int32)

    def _split(v, fill):
        a = jnp.pad(
            v[:EA].reshape(NS, KA, CH),
            ((0, 0), (0, KMAX - KA), (0, 0)),
            constant_values=fill,
        )
        b = jnp.pad(
            v[EA:].reshape(NS, EPB),
            ((0, 0), (0, KMAX * CH - EPB)),
            constant_values=fill,
        ).reshape(NS, KMAX, CH)
        return jnp.concatenate([a, b], axis=0)

    srcw = _split(src, 0)
    dstw = _split(dst, N)
    padd = E_PAD - E
    dstwd = jnp.concatenate([dst, jnp.full((padd,), N, jnp.int32)]).reshape(
        NW, K, CH
    )
    zacc = jnp.zeros((N_ACC, D), jnp.float32)
    ones = jnp.ones((CH, D), jnp.float32)

    hs1, hn1 = _lin2(x, W1_self, W1_nbr, b1)
    degp = _deg(dstwd, zacc, ones)
    hn1, degp = lax.optimization_barrier((hn1, degp))
    parts1 = _agg(hn1, srcw, dstw, zacc)
    d0, d1 = degp[0, :N], degp[1, :N]
    h1, hs2, hn2 = _comb_lin2(
        hs1, parts1[0, :N], parts1[1, :N], d0, d1, W2_self, W2_nbr, b2
    )
    parts2 = _agg(hn2, srcw, dstw, zacc)
    h2 = _comb(hs2, parts2[0, :N], parts2[1, :N], d0, d1)
    return (x, h1, h2)


# speedup vs baseline: 1.0964x; 1.0123x over previous
import jax
import jax.numpy as jnp
from jax import lax
from jax.experimental import pallas as pl
from jax.experimental.pallas import tpu as pltpu
from jax.experimental.pallas import tpu_sc as plsc

N = 10000
D = 128
E = 320000

NC = 2
NS = 16
NW = NC * NS
CH = 128
K = -(-(E // NW) // CH)
KA = 79
KB = 78
KMAX = max(KA, KB)
EA = NS * KA * CH
EPB = -(-(E - EA) // NS)
EPW = K * CH
E_PAD = EPW * NW

RPT = -(-(N + 1) // (NS * 8)) * 8
N_ACC = RPT * NS

_MESH = plsc.VectorSubcoreMesh(
    core_axis_name="c", subcore_axis_name="s", num_cores=NC, num_subcores=NS
)


def _agg_body(hn_hbm, srcw, dstw, zacc, parts, src_v, dst_v, rows_v, acc_sh,
              sem):
    c = lax.axis_index("c")
    s = lax.axis_index("s")
    wid = c * NS + s
    r0 = s * RPT
    pltpu.sync_copy(zacc.at[pl.ds(r0, RPT)], acc_sh.at[pl.ds(r0, RPT)])
    pltpu.sync_copy(srcw.at[wid], src_v)
    pltpu.sync_copy(dstw.at[wid], dst_v)
    plsc.subcore_barrier()

    def step(j, carry):
        pltpu.async_copy(hn_hbm.at[src_v.at[j]], rows_v, sem).wait()
        pltpu.sync_copy(rows_v, acc_sh.at[dst_v.at[j]], add=True)
        return carry

    @pl.when(c == 0)
    def _():
        lax.fori_loop(0, KA, step, 0)

    @pl.when(c == 1)
    def _():
        lax.fori_loop(0, KB, step, 0)

    plsc.subcore_barrier()
    pltpu.sync_copy(acc_sh.at[pl.ds(r0, RPT)], parts.at[c, pl.ds(r0, RPT)])


_agg = pl.kernel(
    _agg_body,
    out_type=jax.ShapeDtypeStruct((NC, N_ACC, D), jnp.float32),
    mesh=_MESH,
    scratch_types=[
        pltpu.VMEM((KMAX, CH), jnp.int32),
        pltpu.VMEM((KMAX, CH), jnp.int32),
        pltpu.VMEM((CH, D), jnp.float32),
        pltpu.VMEM_SHARED((N_ACC, D), jnp.float32),
        pltpu.SemaphoreType.DMA,
    ],
)


def _deg_body(dstw, zdeg, ones_hbm, degp, dst_v, ones_v, deg_sh):
    c = lax.axis_index("c")
    s = lax.axis_index("s")
    wid = c * NS + s
    r0 = s * RPT
    pltpu.sync_copy(zdeg.at[pl.ds(r0, RPT)], deg_sh.at[pl.ds(r0, RPT)])
    pltpu.sync_copy(dstw.at[wid], dst_v)
    pltpu.sync_copy(ones_hbm, ones_v)
    plsc.subcore_barrier()

    def step(j, carry):
        pltpu.sync_copy(ones_v, deg_sh.at[dst_v.at[j]], add=True)
        return carry

    lax.fori_loop(0, K, step, 0)
    plsc.subcore_barrier()
    pltpu.sync_copy(deg_sh.at[pl.ds(r0, RPT)], degp.at[c, pl.ds(r0, RPT)])


_deg = pl.kernel(
    _deg_body,
    out_type=jax.ShapeDtypeStruct((NC, N_ACC, D), jnp.float32),
    mesh=_MESH,
    scratch_types=[
        pltpu.VMEM((K, CH), jnp.int32),
        pltpu.VMEM((CH, D), jnp.float32),
        pltpu.VMEM_SHARED((N_ACC, D), jnp.float32),
    ],
)


BM = 400


def _lin2_tc(h_ref, ws_ref, wn_ref, b_ref, hs_ref, hn_ref):
    h = h_ref[...]
    hs_ref[...] = (
        jnp.dot(h, ws_ref[...], preferred_element_type=jnp.float32) + b_ref[...]
    )
    hn_ref[...] = jnp.dot(h, wn_ref[...], preferred_element_type=jnp.float32)


def _lin2(h, Ws, Wn, b):
    return pl.pallas_call(
        _lin2_tc,
        grid=(N // BM,),
        in_specs=[
            pl.BlockSpec((BM, D), lambda i: (i, 0)),
            pl.BlockSpec((D, D), lambda i: (0, 0)),
            pl.BlockSpec((D, D), lambda i: (0, 0)),
            pl.BlockSpec((1, D), lambda i: (0, 0)),
        ],
        out_specs=[
            pl.BlockSpec((BM, D), lambda i: (i, 0)),
            pl.BlockSpec((BM, D), lambda i: (i, 0)),
        ],
        out_shape=[jax.ShapeDtypeStruct((N, D), jnp.float32)] * 2,
    )(h, Ws, Wn, b.reshape(1, D))


def _mean_relu(hs, p0, p1, d0, d1):
    deg = jnp.maximum((d0 + d1)[:, 0:1], 1.0)
    return jnp.maximum(hs + (p0 + p1) / deg, 0.0)


def _comb_lin2_tc(hs_ref, p0_ref, p1_ref, d0_ref, d1_ref, ws_ref, wn_ref, b_ref,
                  h1_ref, hs2_ref, hn2_ref):
    h1 = _mean_relu(hs_ref[...], p0_ref[...], p1_ref[...], d0_ref[...], d1_ref[...])
    h1_ref[...] = h1
    hs2_ref[...] = (
        jnp.dot(h1, ws_ref[...], preferred_element_type=jnp.float32) + b_ref[...]
    )
    hn2_ref[...] = jnp.dot(h1, wn_ref[...], preferred_element_type=jnp.float32)


def _comb_lin2(hs, p0, p1, d0, d1, Ws, Wn, b):
    return pl.pallas_call(
        _comb_lin2_tc,
        grid=(N // BM,),
        in_specs=[
            pl.BlockSpec((BM, D), lambda i: (i, 0)),
            pl.BlockSpec((BM, D), lambda i: (i, 0)),
            pl.BlockSpec((BM, D), lambda i: (i, 0)),
            pl.BlockSpec((BM, D), lambda i: (i, 0)),
            pl.BlockSpec((BM, D), lambda i: (i, 0)),
            pl.BlockSpec((D, D), lambda i: (0, 0)),
            pl.BlockSpec((D, D), lambda i: (0, 0)),
            pl.BlockSpec((1, D), lambda i: (0, 0)),
        ],
        out_specs=[
            pl.BlockSpec((BM, D), lambda i: (i, 0)),
            pl.BlockSpec((BM, D), lambda i: (i, 0)),
            pl.BlockSpec((BM, D), lambda i: (i, 0)),
        ],
        out_shape=[jax.ShapeDtypeStruct((N, D), jnp.float32)] * 3,
    )(hs, p0, p1, d0, d1, Ws, Wn, b.reshape(1, D))


def _comb_tc(hs_ref, p0_ref, p1_ref, d0_ref, d1_ref, out_ref):
    out_ref[...] = _mean_relu(
        hs_ref[...], p0_ref[...], p1_ref[...], d0_ref[...], d1_ref[...]
    )


def _comb(hs, p0, p1, d0, d1):
    return pl.pallas_call(
        _comb_tc,
        grid=(N // BM,),
        in_specs=[
            pl.BlockSpec((BM, D), lambda i: (i, 0)),
            pl.BlockSpec((BM, D), lambda i: (i, 0)),
            pl.BlockSpec((BM, D), lambda i: (i, 0)),
            pl.BlockSpec((BM, D), lambda i: (i, 0)),
            pl.BlockSpec((BM, D), lambda i: (i, 0)),
        ],
        out_specs=pl.BlockSpec((BM, D), lambda i: (i, 0)),
        out_shape=jax.ShapeDtypeStruct((N, D), jnp.float32),
    )(hs, p0, p1, d0, d1)


def kernel(x, edge_index, W1_self, W1_nbr, b1, W2_self, W2_nbr, b2):
    src = edge_index[0].astype(jnp.int32)
    dst = edge_index[1].astype(jnp.int32)

    def _split(v, fill):
        a = jnp.pad(
            v[:EA].reshape(NS, KA, CH),
            ((0, 0), (0, KMAX - KA), (0, 0)),
            constant_values=fill,
        )
        b = jnp.pad(
            v[EA:].reshape(NS, EPB),
            ((0, 0), (0, KMAX * CH - EPB)),
            constant_values=fill,
        ).reshape(NS, KMAX, CH)
        return jnp.concatenate([a, b], axis=0)

    srcw = _split(src, 0)
    dstw = _split(dst, N)
    padd = E_PAD - E
    dstwd = jnp.concatenate([dst, jnp.full((padd,), N, jnp.int32)]).reshape(
        NW, K, CH
    )
    zacc = jnp.zeros((N_ACC, D), jnp.float32)
    ones = jnp.ones((CH, D), jnp.float32)

    hs1, hn1 = _lin2(x, W1_self, W1_nbr, b1)
    degp = _deg(dstwd, zacc, ones)
    hn1, degp = lax.optimization_barrier((hn1, degp))
    parts1 = _agg(hn1, srcw, dstw, zacc)
    d0, d1 = degp[0, :N], degp[1, :N]
    h1, hs2, hn2 = _comb_lin2(
        hs1, parts1[0, :N], parts1[1, :N], d0, d1, W2_self, W2_nbr, b2
    )
    parts2 = _agg(hn2, srcw, dstw, zacc)
    h2 = _comb(hs2, parts2[0, :N], parts2[1, :N], d0, d1)
    return (x, h1, h2)


# speedup vs baseline: 1.1190x; 1.0206x over previous
import jax
import jax.numpy as jnp
from jax import lax
from jax.experimental import pallas as pl
from jax.experimental.pallas import tpu as pltpu
from jax.experimental.pallas import tpu_sc as plsc

N = 10000
D = 128
E = 320000

NC = 2
NS = 16
NW = NC * NS
CH = 128
K = -(-(E // NW) // CH)
KA = 85
KB = 72
KMAX = max(KA, KB)
EA = NS * KA * CH
EPB = -(-(E - EA) // NS)
EPW = K * CH
E_PAD = EPW * NW

RPT = -(-(N + 1) // (NS * 8)) * 8
N_ACC = RPT * NS

_MESH = plsc.VectorSubcoreMesh(
    core_axis_name="c", subcore_axis_name="s", num_cores=NC, num_subcores=NS
)


def _agg_body(hn_hbm, srcw, dstw, zacc, parts, src_v, dst_v, rows_v, acc_sh,
              sem):
    c = lax.axis_index("c")
    s = lax.axis_index("s")
    wid = c * NS + s
    r0 = s * RPT
    pltpu.sync_copy(zacc.at[pl.ds(r0, RPT)], acc_sh.at[pl.ds(r0, RPT)])
    pltpu.sync_copy(srcw.at[wid], src_v)
    pltpu.sync_copy(dstw.at[wid], dst_v)
    plsc.subcore_barrier()

    def step(j, carry):
        pltpu.async_copy(hn_hbm.at[src_v.at[j]], rows_v, sem).wait()
        pltpu.sync_copy(rows_v, acc_sh.at[dst_v.at[j]], add=True)
        return carry

    @pl.when(c == 0)
    def _():
        lax.fori_loop(0, KA, step, 0)

    @pl.when(c == 1)
    def _():
        lax.fori_loop(0, KB, step, 0)

    plsc.subcore_barrier()
    pltpu.sync_copy(acc_sh.at[pl.ds(r0, RPT)], parts.at[c, pl.ds(r0, RPT)])


_agg = pl.kernel(
    _agg_body,
    out_type=jax.ShapeDtypeStruct((NC, N_ACC, D), jnp.float32),
    mesh=_MESH,
    scratch_types=[
        pltpu.VMEM((KMAX, CH), jnp.int32),
        pltpu.VMEM((KMAX, CH), jnp.int32),
        pltpu.VMEM((CH, D), jnp.float32),
        pltpu.VMEM_SHARED((N_ACC, D), jnp.float32),
        pltpu.SemaphoreType.DMA,
    ],
)


def _deg_body(dstw, zdeg, ones_hbm, degp, dst_v, ones_v, deg_sh):
    c = lax.axis_index("c")
    s = lax.axis_index("s")
    wid = c * NS + s
    r0 = s * RPT
    pltpu.sync_copy(zdeg.at[pl.ds(r0, RPT)], deg_sh.at[pl.ds(r0, RPT)])
    pltpu.sync_copy(dstw.at[wid], dst_v)
    pltpu.sync_copy(ones_hbm, ones_v)
    plsc.subcore_barrier()

    def step(j, carry):
        pltpu.sync_copy(ones_v, deg_sh.at[dst_v.at[j]], add=True)
        return carry

    lax.fori_loop(0, K, step, 0)
    plsc.subcore_barrier()
    pltpu.sync_copy(deg_sh.at[pl.ds(r0, RPT)], degp.at[c, pl.ds(r0, RPT)])


_deg = pl.kernel(
    _deg_body,
    out_type=jax.ShapeDtypeStruct((NC, N_ACC, D), jnp.float32),
    mesh=_MESH,
    scratch_types=[
        pltpu.VMEM((K, CH), jnp.int32),
        pltpu.VMEM((CH, D), jnp.float32),
        pltpu.VMEM_SHARED((N_ACC, D), jnp.float32),
    ],
)


BM = 400


def _lin2_tc(h_ref, ws_ref, wn_ref, b_ref, hs_ref, hn_ref):
    h = h_ref[...]
    hs_ref[...] = (
        jnp.dot(h, ws_ref[...], preferred_element_type=jnp.float32) + b_ref[...]
    )
    hn_ref[...] = jnp.dot(h, wn_ref[...], preferred_element_type=jnp.float32)


def _lin2(h, Ws, Wn, b):
    return pl.pallas_call(
        _lin2_tc,
        grid=(N // BM,),
        in_specs=[
            pl.BlockSpec((BM, D), lambda i: (i, 0)),
            pl.BlockSpec((D, D), lambda i: (0, 0)),
            pl.BlockSpec((D, D), lambda i: (0, 0)),
            pl.BlockSpec((1, D), lambda i: (0, 0)),
        ],
        out_specs=[
            pl.BlockSpec((BM, D), lambda i: (i, 0)),
            pl.BlockSpec((BM, D), lambda i: (i, 0)),
        ],
        out_shape=[jax.ShapeDtypeStruct((N, D), jnp.float32)] * 2,
    )(h, Ws, Wn, b.reshape(1, D))


def _mean_relu(hs, p0, p1, d0, d1):
    deg = jnp.maximum((d0 + d1)[:, 0:1], 1.0)
    return jnp.maximum(hs + (p0 + p1) / deg, 0.0)


def _comb_lin2_tc(hs_ref, p0_ref, p1_ref, d0_ref, d1_ref, ws_ref, wn_ref, b_ref,
                  h1_ref, hs2_ref, hn2_ref):
    h1 = _mean_relu(hs_ref[...], p0_ref[...], p1_ref[...], d0_ref[...], d1_ref[...])
    h1_ref[...] = h1
    hs2_ref[...] = (
        jnp.dot(h1, ws_ref[...], preferred_element_type=jnp.float32) + b_ref[...]
    )
    hn2_ref[...] = jnp.dot(h1, wn_ref[...], preferred_element_type=jnp.float32)


def _comb_lin2(hs, p0, p1, d0, d1, Ws, Wn, b):
    return pl.pallas_call(
        _comb_lin2_tc,
        grid=(N // BM,),
        in_specs=[
            pl.BlockSpec((BM, D), lambda i: (i, 0)),
            pl.BlockSpec((BM, D), lambda i: (i, 0)),
            pl.BlockSpec((BM, D), lambda i: (i, 0)),
            pl.BlockSpec((BM, D), lambda i: (i, 0)),
            pl.BlockSpec((BM, D), lambda i: (i, 0)),
            pl.BlockSpec((D, D), lambda i: (0, 0)),
            pl.BlockSpec((D, D), lambda i: (0, 0)),
            pl.BlockSpec((1, D), lambda i: (0, 0)),
        ],
        out_specs=[
            pl.BlockSpec((BM, D), lambda i: (i, 0)),
            pl.BlockSpec((BM, D), lambda i: (i, 0)),
            pl.BlockSpec((BM, D), lambda i: (i, 0)),
        ],
        out_shape=[jax.ShapeDtypeStruct((N, D), jnp.float32)] * 3,
    )(hs, p0, p1, d0, d1, Ws, Wn, b.reshape(1, D))


def _comb_tc(hs_ref, p0_ref, p1_ref, d0_ref, d1_ref, out_ref):
    out_ref[...] = _mean_relu(
        hs_ref[...], p0_ref[...], p1_ref[...], d0_ref[...], d1_ref[...]
    )


def _comb(hs, p0, p1, d0, d1):
    return pl.pallas_call(
        _comb_tc,
        grid=(N // BM,),
        in_specs=[
            pl.BlockSpec((BM, D), lambda i: (i, 0)),
            pl.BlockSpec((BM, D), lambda i: (i, 0)),
            pl.BlockSpec((BM, D), lambda i: (i, 0)),
            pl.BlockSpec((BM, D), lambda i: (i, 0)),
            pl.BlockSpec((BM, D), lambda i: (i, 0)),
        ],
        out_specs=pl.BlockSpec((BM, D), lambda i: (i, 0)),
        out_shape=jax.ShapeDtypeStruct((N, D), jnp.float32),
    )(hs, p0, p1, d0, d1)


def kernel(x, edge_index, W1_self, W1_nbr, b1, W2_self, W2_nbr, b2):
    src = edge_index[0].astype(jnp.int32)
    dst = edge_index[1].astype(jnp.int32)

    def _split(v, fill):
        a = jnp.pad(
            v[:EA].reshape(NS, KA, CH),
            ((0, 0), (0, KMAX - KA), (0, 0)),
            constant_values=fill,
        )
        b = jnp.pad(
            v[EA:].reshape(NS, EPB),
            ((0, 0), (0, KMAX * CH - EPB)),
            constant_values=fill,
        ).reshape(NS, KMAX, CH)
        return jnp.concatenate([a, b], axis=0)

    srcw = _split(src, 0)
    dstw = _split(dst, N)
    padd = E_PAD - E
    dstwd = jnp.concatenate([dst, jnp.full((padd,), N, jnp.int32)]).reshape(
        NW, K, CH
    )
    zacc = jnp.zeros((N_ACC, D), jnp.float32)
    ones = jnp.ones((CH, D), jnp.float32)

    hs1, hn1 = _lin2(x, W1_self, W1_nbr, b1)
    degp = _deg(dstwd, zacc, ones)
    hn1, degp = lax.optimization_barrier((hn1, degp))
    parts1 = _agg(hn1, srcw, dstw, zacc)
    d0, d1 = degp[0, :N], degp[1, :N]
    h1, hs2, hn2 = _comb_lin2(
        hs1, parts1[0, :N], parts1[1, :N], d0, d1, W2_self, W2_nbr, b2
    )
    parts2 = _agg(hn2, srcw, dstw, zacc)
    h2 = _comb(hs2, parts2[0, :N], parts2[1, :N], d0, d1)
    return (x, h1, h2)
